# baseline (speedup 1.0000x reference)
import functools

import jax
import jax.numpy as jnp
from jax.experimental import pallas as pl
from jax.experimental.pallas import tpu as pltpu

_BN_EPS = 1e-5
_PACK = 8
_GROUPS = 1
_C = 32
_CIN_PAD = 8


def _conv_kernel(xp_ref, w1c_ref, b1_ref, w2c_ref, b2_ref, w3c_ref, b3_ref,
                 o_ref, act_ref, act2a_ref, act2b_ref,
                 w1s_ref, w2s_ref, w3s_ref, *, H, W):
    Ho, Wo = H // 2, W // 2
    L = _PACK * _C
    HW = H * W

    @pl.when(pl.program_id(0) == 0)
    def _build_weights():
        w1s_ref[...] = jnp.zeros_like(w1s_ref)
        w2s_ref[...] = jnp.zeros_like(w2s_ref)
        w3s_ref[...] = jnp.zeros_like(w3s_ref)
        for b in range(_PACK):
            lo = b * _C
            for k in range(9):
                w1s_ref[k * _PACK + b:k * _PACK + b + 1, lo:lo + _C] = (
                    w1c_ref[k:k + 1, :])
                w2s_ref[k, lo:lo + _C, lo:lo + _C] = w2c_ref[k]
                w3s_ref[k, lo:lo + _C, lo:lo + _C] = w3c_ref[k]

    for i in range(_GROUPS):
        xp = xp_ref[i].reshape(H + 2, W + 2, _PACK)
        pieces = [xp[dh:dh + H, dw:dw + W, :]
                  for dh in range(3) for dw in range(3)]
        p1 = jnp.concatenate(pieces, axis=-1).reshape(HW, 9 * _PACK)
        y = jnp.dot(p1, w1s_ref[...], preferred_element_type=jnp.float32)
        y = jnp.maximum(y + b1_ref[...], 0.0).astype(jnp.bfloat16)

        act_ref[0:1, :, :] = jnp.zeros((1, W + 2, L), jnp.bfloat16)
        act_ref[H + 1:H + 2, :, :] = jnp.zeros((1, W + 2, L), jnp.bfloat16)
        act_ref[:, 0:1, :] = jnp.zeros((H + 2, 1, L), jnp.bfloat16)
        act_ref[:, W + 1:W + 2, :] = jnp.zeros((H + 2, 1, L), jnp.bfloat16)
        act_ref[1:H + 1, 1:W + 1, :] = y.reshape(H, W, L)

        y = None
        for k in range(9):
            dh, dw = divmod(k, 3)
            t = jnp.dot(act_ref[dh:dh + H, dw:dw + W, :].reshape(HW, L),
                        w2s_ref[k], preferred_element_type=jnp.float32)
            y = t if y is None else y + t
        y = jnp.maximum(y + b2_ref[...], 0.0)
        y3 = y.reshape(H, W, L)
        for r in (act2a_ref, act2b_ref):
            r[0:1, :, :] = jnp.zeros((1, W + 2, 128), jnp.float32)
            r[H + 1:H + 2, :, :] = jnp.zeros((1, W + 2, 128), jnp.float32)
            r[:, 0:1, :] = jnp.zeros((H + 2, 1, 128), jnp.float32)
            r[:, W + 1:W + 2, :] = jnp.zeros((H + 2, 1, 128), jnp.float32)
        act2a_ref[1:H + 1, 1:W + 1, :] = y3[:, :, 0:128]
        act2b_ref[1:H + 1, 1:W + 1, :] = y3[:, :, 128:L]

        y = None
        for k in range(9):
            dh, dw = divmod(k, 3)
            v = jnp.concatenate([
                act2a_ref[pl.ds(dh, Ho, 2), pl.ds(dw, Wo, 2), :],
                act2b_ref[pl.ds(dh, Ho, 2), pl.ds(dw, Wo, 2), :],
            ], axis=-1).reshape(Ho * Wo, L).astype(jnp.bfloat16)
            t = jnp.dot(v, w3s_ref[k], preferred_element_type=jnp.float32)
            y = t if y is None else y + t
        yo = jnp.maximum(y + b3_ref[...], 0.0).astype(jnp.bfloat16)
        for b in range(_PACK):
            o_ref[i, b, :, :] = yo[:, b * _C:(b + 1) * _C]


def _conv_stage(xp, w1c, b1p, w2c, b2p, w3c, b3p, *, H, W):
    G2 = xp.shape[0]
    Ho, Wo = H // 2, W // 2
    L = _PACK * _C
    return pl.pallas_call(
        functools.partial(_conv_kernel, H=H, W=W),
        out_shape=jax.ShapeDtypeStruct(
            (G2, _GROUPS, _PACK, Ho * Wo, _C), jnp.bfloat16),
        grid=(G2,),
        in_specs=[
            pl.BlockSpec((None, _GROUPS, H + 2, (W + 2) * _PACK),
                         lambda g: (g, 0, 0, 0)),
            pl.BlockSpec((9, _C), lambda g: (0, 0)),
            pl.BlockSpec((1, L), lambda g: (0, 0)),
            pl.BlockSpec((9, _C, _C), lambda g: (0, 0, 0)),
            pl.BlockSpec((1, L), lambda g: (0, 0)),
            pl.BlockSpec((9, _C, _C), lambda g: (0, 0, 0)),
            pl.BlockSpec((1, L), lambda g: (0, 0)),
        ],
        out_specs=pl.BlockSpec((None, _GROUPS, _PACK, Ho * Wo, _C),
                               lambda g: (g, 0, 0, 0, 0)),
        scratch_shapes=[
            pltpu.VMEM((H + 2, W + 2, L), jnp.bfloat16),
            pltpu.VMEM((H + 2, W + 2, 128), jnp.float32),
            pltpu.VMEM((H + 2, W + 2, 128), jnp.float32),
            pltpu.VMEM((9 * _PACK, L), jnp.bfloat16),
            pltpu.VMEM((9, L, L), jnp.bfloat16),
            pltpu.VMEM((9, L, L), jnp.bfloat16),
        ],
        compiler_params=pltpu.CompilerParams(
            dimension_semantics=("arbitrary",),
            vmem_limit_bytes=32 * 1024 * 1024),
    )(xp, w1c, b1p, w2c, b2p, w3c, b3p)


def _dense_kernel(x_ref, w1_ref, b1_ref, g1_ref, bt1_ref,
                  w2_ref, b2_ref, g2_ref, bt2_ref,
                  wh_ref, bh_ref, o_ref):
    def dense_block(x, w, b_r, g_r, bt_r):
        y = jnp.dot(x, w, preferred_element_type=jnp.float32) + b_r[...]
        m = jnp.mean(y, axis=0, keepdims=True)
        v = jnp.mean(jnp.square(y - m), axis=0, keepdims=True)
        y = (y - m) * jax.lax.rsqrt(v + _BN_EPS) * g_r[...] + bt_r[...]
        return jnp.maximum(y, 0.0).astype(jnp.bfloat16)

    h = dense_block(x_ref[...], w1_ref[...].astype(jnp.bfloat16),
                    b1_ref, g1_ref, bt1_ref)
    h = dense_block(h, w2_ref[...], b2_ref, g2_ref, bt2_ref)
    o_ref[...] = (jnp.dot(h, wh_ref[...], preferred_element_type=jnp.float32)
                  + bh_ref[...])


def _dense_stage(flat, fc1_w, fc1_b, fc1_g, fc1_beta,
                 fc2_w, fc2_b, fc2_g, fc2_beta, head_w, head_b):
    N = flat.shape[0]
    M = head_w.shape[1]
    return pl.pallas_call(
        _dense_kernel,
        out_shape=jax.ShapeDtypeStruct((N, M), jnp.float32),
        compiler_params=pltpu.CompilerParams(
            vmem_limit_bytes=48 * 1024 * 1024),
    )(flat, fc1_w, fc1_b, fc1_g, fc1_beta,
      fc2_w, fc2_b, fc2_g, fc2_beta, head_w, head_b)


def kernel(x_nchw, sel, c1_w, c1_b, c2_w, c2_b, c3_w, c3_b,
           fc1_w, fc1_b, fc1_g, fc1_beta,
           fc2_w, fc2_b, fc2_g, fc2_beta, head_w, head_b):
    N, _, H, W = x_nchw.shape
    Ho, Wo = H // 2, W // 2
    G = N // _PACK
    L = _PACK * _C
    latent = head_w.shape[1] // 2

    w1c = c1_w.reshape(9, _CIN_PAD, _C)[:, 0, :].astype(jnp.bfloat16)
    w2c = c2_w.reshape(9, _C, _C).astype(jnp.bfloat16)
    w3c = c3_w.reshape(9, _C, _C).astype(jnp.bfloat16)
    b1p = jnp.tile(c1_b, (1, _PACK))
    b2p = jnp.tile(c2_b, (1, _PACK))
    b3p = jnp.tile(c3_b, (1, _PACK))

    xg = x_nchw.reshape(G, _PACK, H, W).transpose(0, 2, 3, 1)
    xp = (jnp.pad(xg, ((0, 0), (1, 1), (1, 1), (0, 0)))
          .astype(jnp.bfloat16)
          .reshape(G // _GROUPS, _GROUPS, H + 2, (W + 2) * _PACK))

    conv = _conv_stage(xp, w1c, b1p, w2c, b2p, w3c, b3p, H=H, W=W)

    flat = conv.reshape(N, Ho * Wo * _C)

    out = _dense_stage(flat, fc1_w, fc1_b, fc1_g, fc1_beta,
                       fc2_w.astype(jnp.bfloat16), fc2_b, fc2_g, fc2_beta,
                       head_w.astype(jnp.bfloat16), head_b)
    return out[:, :latent], out[:, latent:]


# speedup vs baseline: 1.0037x; 1.0037x over previous
import functools

import jax
import jax.numpy as jnp
from jax.experimental import pallas as pl
from jax.experimental.pallas import tpu as pltpu

_BN_EPS = 1e-5
_PACK = 8
_GROUPS = 1
_C = 32
_CIN_PAD = 8


def _conv_kernel(xp_ref, w1c_ref, b1_ref, w2c_ref, b2_ref, w3c_ref, b3_ref,
                 o_ref, act_ref, act2a_ref, act2b_ref,
                 w1s_ref, w2s_ref, w3s_ref, *, H, W):
    Ho, Wo = H // 2, W // 2
    L = _PACK * _C
    HW = H * W

    @pl.when(pl.program_id(0) == 0)
    def _build_weights():
        w1s_ref[...] = jnp.zeros_like(w1s_ref)
        w2s_ref[...] = jnp.zeros_like(w2s_ref)
        w3s_ref[...] = jnp.zeros_like(w3s_ref)
        for b in range(_PACK):
            lo = b * _C
            for k in range(9):
                w1s_ref[k * _PACK + b:k * _PACK + b + 1, lo:lo + _C] = (
                    w1c_ref[k:k + 1, :])
                w2s_ref[k, lo:lo + _C, lo:lo + _C] = w2c_ref[k]
                w3s_ref[k, lo:lo + _C, lo:lo + _C] = w3c_ref[k]
        act_ref[0:1, :, :] = jnp.zeros((1, W + 2, L), jnp.bfloat16)
        act_ref[H + 1:H + 2, :, :] = jnp.zeros((1, W + 2, L), jnp.bfloat16)
        act_ref[:, 0:1, :] = jnp.zeros((H + 2, 1, L), jnp.bfloat16)
        act_ref[:, W + 1:W + 2, :] = jnp.zeros((H + 2, 1, L), jnp.bfloat16)
        for r in (act2a_ref, act2b_ref):
            r[0:1, :, :] = jnp.zeros((1, W + 2, 128), jnp.float32)
            r[H + 1:H + 2, :, :] = jnp.zeros((1, W + 2, 128), jnp.float32)
            r[:, 0:1, :] = jnp.zeros((H + 2, 1, 128), jnp.float32)
            r[:, W + 1:W + 2, :] = jnp.zeros((H + 2, 1, 128), jnp.float32)

    for i in range(_GROUPS):
        xp = xp_ref[i].reshape(H + 2, W + 2, _PACK)
        pieces = [xp[dh:dh + H, dw:dw + W, :]
                  for dh in range(3) for dw in range(3)]
        p1 = jnp.concatenate(pieces, axis=-1).reshape(HW, 9 * _PACK)
        y = jnp.dot(p1, w1s_ref[...], preferred_element_type=jnp.float32)
        y = jnp.maximum(y + b1_ref[...], 0.0).astype(jnp.bfloat16)

        act_ref[1:H + 1, 1:W + 1, :] = y.reshape(H, W, L)

        y = None
        for k in range(9):
            dh, dw = divmod(k, 3)
            t = jnp.dot(act_ref[dh:dh + H, dw:dw + W, :].reshape(HW, L),
                        w2s_ref[k], preferred_element_type=jnp.float32)
            y = t if y is None else y + t
        y = jnp.maximum(y + b2_ref[...], 0.0)
        y3 = y.reshape(H, W, L)
        act2a_ref[1:H + 1, 1:W + 1, :] = y3[:, :, 0:128]
        act2b_ref[1:H + 1, 1:W + 1, :] = y3[:, :, 128:L]

        y = None
        for k in range(9):
            dh, dw = divmod(k, 3)
            v = jnp.concatenate([
                act2a_ref[pl.ds(dh, Ho, 2), pl.ds(dw, Wo, 2), :],
                act2b_ref[pl.ds(dh, Ho, 2), pl.ds(dw, Wo, 2), :],
            ], axis=-1).reshape(Ho * Wo, L).astype(jnp.bfloat16)
            t = jnp.dot(v, w3s_ref[k], preferred_element_type=jnp.float32)
            y = t if y is None else y + t
        yo = jnp.maximum(y + b3_ref[...], 0.0).astype(jnp.bfloat16)
        for b in range(_PACK):
            o_ref[i, b, :, :] = yo[:, b * _C:(b + 1) * _C]


def _conv_stage(xp, w1c, b1p, w2c, b2p, w3c, b3p, *, H, W):
    G2 = xp.shape[0]
    Ho, Wo = H // 2, W // 2
    L = _PACK * _C
    return pl.pallas_call(
        functools.partial(_conv_kernel, H=H, W=W),
        out_shape=jax.ShapeDtypeStruct(
            (G2, _GROUPS, _PACK, Ho * Wo, _C), jnp.bfloat16),
        grid=(G2,),
        in_specs=[
            pl.BlockSpec((None, _GROUPS, H + 2, (W + 2) * _PACK),
                         lambda g: (g, 0, 0, 0)),
            pl.BlockSpec((9, _C), lambda g: (0, 0)),
            pl.BlockSpec((1, L), lambda g: (0, 0)),
            pl.BlockSpec((9, _C, _C), lambda g: (0, 0, 0)),
            pl.BlockSpec((1, L), lambda g: (0, 0)),
            pl.BlockSpec((9, _C, _C), lambda g: (0, 0, 0)),
            pl.BlockSpec((1, L), lambda g: (0, 0)),
        ],
        out_specs=pl.BlockSpec((None, _GROUPS, _PACK, Ho * Wo, _C),
                               lambda g: (g, 0, 0, 0, 0)),
        scratch_shapes=[
            pltpu.VMEM((H + 2, W + 2, L), jnp.bfloat16),
            pltpu.VMEM((H + 2, W + 2, 128), jnp.float32),
            pltpu.VMEM((H + 2, W + 2, 128), jnp.float32),
            pltpu.VMEM((9 * _PACK, L), jnp.bfloat16),
            pltpu.VMEM((9, L, L), jnp.bfloat16),
            pltpu.VMEM((9, L, L), jnp.bfloat16),
        ],
        compiler_params=pltpu.CompilerParams(
            dimension_semantics=("arbitrary",),
            vmem_limit_bytes=32 * 1024 * 1024),
    )(xp, w1c, b1p, w2c, b2p, w3c, b3p)


def _dense_kernel(x_ref, w1_ref, b1_ref, g1_ref, bt1_ref,
                  w2_ref, b2_ref, g2_ref, bt2_ref,
                  wh_ref, bh_ref, o_ref):
    def dense_block(x, w, b_r, g_r, bt_r):
        y = jnp.dot(x, w, preferred_element_type=jnp.float32) + b_r[...]
        m = jnp.mean(y, axis=0, keepdims=True)
        v = jnp.mean(jnp.square(y - m), axis=0, keepdims=True)
        y = (y - m) * jax.lax.rsqrt(v + _BN_EPS) * g_r[...] + bt_r[...]
        return jnp.maximum(y, 0.0).astype(jnp.bfloat16)

    h = dense_block(x_ref[...], w1_ref[...].astype(jnp.bfloat16),
                    b1_ref, g1_ref, bt1_ref)
    h = dense_block(h, w2_ref[...], b2_ref, g2_ref, bt2_ref)
    o_ref[...] = (jnp.dot(h, wh_ref[...], preferred_element_type=jnp.float32)
                  + bh_ref[...])


def _dense_stage(flat, fc1_w, fc1_b, fc1_g, fc1_beta,
                 fc2_w, fc2_b, fc2_g, fc2_beta, head_w, head_b):
    N = flat.shape[0]
    M = head_w.shape[1]
    return pl.pallas_call(
        _dense_kernel,
        out_shape=jax.ShapeDtypeStruct((N, M), jnp.float32),
        compiler_params=pltpu.CompilerParams(
            vmem_limit_bytes=48 * 1024 * 1024),
    )(flat, fc1_w, fc1_b, fc1_g, fc1_beta,
      fc2_w, fc2_b, fc2_g, fc2_beta, head_w, head_b)


def kernel(x_nchw, sel, c1_w, c1_b, c2_w, c2_b, c3_w, c3_b,
           fc1_w, fc1_b, fc1_g, fc1_beta,
           fc2_w, fc2_b, fc2_g, fc2_beta, head_w, head_b):
    N, _, H, W = x_nchw.shape
    Ho, Wo = H // 2, W // 2
    G = N // _PACK
    L = _PACK * _C
    latent = head_w.shape[1] // 2

    w1c = c1_w.reshape(9, _CIN_PAD, _C)[:, 0, :].astype(jnp.bfloat16)
    w2c = c2_w.reshape(9, _C, _C).astype(jnp.bfloat16)
    w3c = c3_w.reshape(9, _C, _C).astype(jnp.bfloat16)
    b1p = jnp.tile(c1_b, (1, _PACK))
    b2p = jnp.tile(c2_b, (1, _PACK))
    b3p = jnp.tile(c3_b, (1, _PACK))

    xg = x_nchw.reshape(G, _PACK, H, W).transpose(0, 2, 3, 1)
    xp = (jnp.pad(xg, ((0, 0), (1, 1), (1, 1), (0, 0)))
          .astype(jnp.bfloat16)
          .reshape(G // _GROUPS, _GROUPS, H + 2, (W + 2) * _PACK))

    conv = _conv_stage(xp, w1c, b1p, w2c, b2p, w3c, b3p, H=H, W=W)

    flat = conv.reshape(N, Ho * Wo * _C)

    out = _dense_stage(flat, fc1_w, fc1_b, fc1_g, fc1_beta,
                       fc2_w.astype(jnp.bfloat16), fc2_b, fc2_g, fc2_beta,
                       head_w.astype(jnp.bfloat16), head_b)
    return out[:, :latent], out[:, latent:]
